# Initial kernel scaffold; baseline (speedup 1.0000x reference)
import jax
import jax.numpy as jnp
from jax.experimental import pallas as pl


def kernel(x_0, W0, W1, b01, b10, rows, cols):
    raise NotImplementedError("write your pallas kernel here")


# speedup vs baseline: 18.7954x; 18.7954x over previous
import functools

import jax
import jax.numpy as jnp
from jax import lax
from jax.experimental import pallas as pl
from jax.experimental.pallas import tpu as pltpu
from jax.experimental.pallas import tpu_sc as plsc

N_NODES = 10000
N_EDGES = 5000
NNZ = 320000
D = 128
NP = 10240
EP = 5120
NW = 32
NCHUNK = 80
CL = 128
NPH = NP // 2
ACC_E2N = NPH + 128
OUT_E2N = NP + 128


def _mesh():
    return plsc.VectorSubcoreMesh(core_axis_name="c", subcore_axis_name="s")


def _rsqrt16(v):
    i = plsc.bitcast(v, jnp.int32)
    i = 0x5F3759DF - lax.shift_right_logical(i, 1)
    y = plsc.bitcast(i, jnp.float32)
    for _ in range(3):
        y = y * (1.5 - 0.5 * v * y * y)
    return y


def _zero_flat(ref, n):
    z = jnp.zeros((16,), jnp.float32)

    def body(i, _):
        ref[pl.ds(i * 16, 16)] = z
        return 0

    lax.fori_loop(0, n // 16, body, 0)


@functools.partial(
    pl.kernel,
    out_type=(jax.ShapeDtypeStruct((NW, 1, NP), jnp.float32),
              jax.ShapeDtypeStruct((NW, 1, EP), jnp.float32)),
    mesh=_mesh(),
    compiler_params=pltpu.CompilerParams(needs_layout_passes=False),
    scratch_types=[
        pltpu.VMEM((NCHUNK, CL), jnp.int32),
        pltpu.VMEM((NCHUNK, CL), jnp.int32),
        pltpu.VMEM((NP,), jnp.float32),
        pltpu.VMEM((EP,), jnp.float32),
    ])
def _s1_counts(ridx_h, cidx_h, cn_out, ce_out, ridx, cidx, cn, ce):
    c = lax.axis_index("c")
    s = lax.axis_index("s")
    wid = s * 2 + c
    pltpu.sync_copy(ridx_h.at[wid], ridx)
    pltpu.sync_copy(cidx_h.at[wid], cidx)
    _zero_flat(cn, NP)
    _zero_flat(ce, EP)
    ones = jnp.ones((16,), jnp.float32)

    def chunk(j, _):
        for g in range(8):
            r = ridx[j, pl.ds(g * 16, 16)]
            cc = cidx[j, pl.ds(g * 16, 16)]
            plsc.addupdate_scatter(cn, [r], ones)
            plsc.addupdate_scatter(ce, [cc], ones)
        return 0

    lax.fori_loop(0, NCHUNK, chunk, 0)
    pltpu.sync_copy(cn, cn_out.at[wid, 0])
    pltpu.sync_copy(ce, ce_out.at[wid, 0])


@functools.partial(
    pl.kernel,
    out_type=(jax.ShapeDtypeStruct((NW, 1, NP), jnp.float32),
              jax.ShapeDtypeStruct((NW, 1, EP), jnp.float32)),
    mesh=_mesh(),
    compiler_params=pltpu.CompilerParams(needs_layout_passes=False),
    scratch_types=[
        pltpu.VMEM((NCHUNK, CL), jnp.int32),
        pltpu.VMEM((NCHUNK, CL), jnp.int32),
        pltpu.VMEM((NP,), jnp.float32),
        pltpu.VMEM((EP,), jnp.float32),
        pltpu.VMEM((NP,), jnp.float32),
        pltpu.VMEM((EP,), jnp.float32),
    ])
def _s2_dsums(ridx_h, cidx_h, cn_h, ce_h, d0_out, d1_out,
              ridx, cidx, nct, ect, d0s, d1s):
    c = lax.axis_index("c")
    s = lax.axis_index("s")
    wid = s * 2 + c
    pltpu.sync_copy(ridx_h.at[wid], ridx)
    pltpu.sync_copy(cidx_h.at[wid], cidx)
    pltpu.sync_copy(cn_h, nct)
    pltpu.sync_copy(ce_h, ect)

    def nbody(i, _):
        v = jnp.maximum(nct[pl.ds(i * 16, 16)], 1.0)
        nct[pl.ds(i * 16, 16)] = _rsqrt16(v)
        return 0

    lax.fori_loop(0, NP // 16, nbody, 0)

    def ebody(i, _):
        v = jnp.maximum(ect[pl.ds(i * 16, 16)], 1.0)
        rs = _rsqrt16(v)
        ect[pl.ds(i * 16, 16)] = rs * rs * rs
        return 0

    lax.fori_loop(0, EP // 16, ebody, 0)
    _zero_flat(d0s, NP)
    _zero_flat(d1s, EP)

    def chunk(j, _):
        for g in range(8):
            r = ridx[j, pl.ds(g * 16, 16)]
            cc = cidx[j, pl.ds(g * 16, 16)]
            ecv = plsc.load_gather(ect, [cc])
            plsc.addupdate_scatter(d0s, [r], ecv)
            ncv = plsc.load_gather(nct, [r])
            plsc.addupdate_scatter(d1s, [cc], ncv)
        return 0

    lax.fori_loop(0, NCHUNK, chunk, 0)
    pltpu.sync_copy(d0s, d0_out.at[wid, 0])
    pltpu.sync_copy(d1s, d1_out.at[wid, 0])


_TBN = NP // NW
_TBE = EP // NW


@functools.partial(
    pl.kernel,
    out_type=(jax.ShapeDtypeStruct((NP, D), jnp.float32),
              jax.ShapeDtypeStruct((EP, D), jnp.float32),
              jax.ShapeDtypeStruct((NP, D), jnp.float32),
              jax.ShapeDtypeStruct((EP, D), jnp.float32)),
    mesh=_mesh(),
    compiler_params=pltpu.CompilerParams(needs_layout_passes=False),
    scratch_types=[
        pltpu.VMEM((_TBN,), jnp.float32),
        pltpu.VMEM((16, D), jnp.float32),
    ])
def _s3_scales(cn_h, ce_h, d0_h, d1_h, nc_b, ec_b, d0i_b, d1i_b, vraw, obuf):
    c = lax.axis_index("c")
    s = lax.axis_index("s")
    wid = s * 2 + c

    def emit(src_h, nrows, base, fn, out_ref):
        pltpu.sync_copy(src_h.at[pl.ds(base, nrows)], vraw.at[pl.ds(0, nrows)])

        def body(r, _):
            v = fn(vraw[pl.ds(r * 16, 16)])
            for l in range(16):
                row = jnp.full((16,), v[l], jnp.float32)
                for k in range(8):
                    obuf[l, pl.ds(k * 16, 16)] = row
            pltpu.sync_copy(obuf, out_ref.at[pl.ds(base + r * 16, 16)])
            return 0

        lax.fori_loop(0, nrows // 16, body, 0)

    def f_nc(v):
        return _rsqrt16(jnp.maximum(v, 1.0))

    def f_ec(v):
        rs = _rsqrt16(jnp.maximum(v, 1.0))
        return rs * rs * rs

    def f_inv(v):
        rs = _rsqrt16(jnp.maximum(v, 1e-20))
        return rs * rs

    emit(cn_h, _TBN, wid * _TBN, f_nc, nc_b)
    emit(d0_h, _TBN, wid * _TBN, f_inv, d0i_b)
    emit(ce_h, _TBE, wid * _TBE, f_ec, ec_b)
    emit(d1_h, _TBE, wid * _TBE, f_inv, d1i_b)


_GRPE = EP // 16


@functools.partial(
    pl.kernel,
    out_type=(jax.ShapeDtypeStruct((EP, D), jnp.float32),
              jax.ShapeDtypeStruct((EP, D), jnp.float32)),
    mesh=_mesh(),
    compiler_params=pltpu.CompilerParams(needs_layout_passes=False),
    scratch_types=[
        pltpu.VMEM((NCHUNK, CL), jnp.int32),
        pltpu.VMEM((NCHUNK, CL), jnp.int32),
        pltpu.VMEM((CL, D), jnp.float32),
        pltpu.VMEM((CL, D), jnp.float32),
        pltpu.VMEM((64, D), jnp.float32),
        pltpu.VMEM_SHARED((EP, D), jnp.float32),
        pltpu.SemaphoreType.DMA,
        pltpu.SemaphoreType.DMA,
    ])
def _seg_n2e(table, gidx_h, sidx_h, out0, out1,
             gidx, sidx, buf0, buf1, zbuf, acc, sem0, sem1):
    c = lax.axis_index("c")
    s = lax.axis_index("s")
    wid = s * 2 + c
    pltpu.sync_copy(gidx_h.at[wid], gidx)
    pltpu.sync_copy(sidx_h.at[wid], sidx)
    z16 = jnp.zeros((16,), jnp.float32)

    def zb(i, _):
        zbuf[i // 8, pl.ds((i % 8) * 16, 16)] = z16
        return 0

    lax.fori_loop(0, 64 * 8, zb, 0)
    base = s * _GRPE
    for zi in range(_GRPE // 64):
        pltpu.sync_copy(zbuf, acc.at[pl.ds(base + zi * 64, 64)])
    plsc.subcore_barrier()

    def start(j, buf, sem):
        pltpu.async_copy(table.at[gidx.at[j]], buf, sem)

    def wait(j, buf, sem):
        pltpu.make_async_copy(table.at[gidx.at[j]], buf, sem).wait()

    def scat(j, buf):
        pltpu.sync_copy(buf, acc.at[sidx.at[j]], add=True)

    start(0, buf0, sem0)
    start(1, buf1, sem1)

    def body(i, _):
        j = i * 2
        wait(j, buf0, sem0)
        scat(j, buf0)
        start(j + 2, buf0, sem0)
        wait(j + 1, buf1, sem1)
        scat(j + 1, buf1)
        start(j + 3, buf1, sem1)
        return 0

    lax.fori_loop(0, NCHUNK // 2 - 1, body, 0)
    j = NCHUNK - 2
    wait(j, buf0, sem0)
    scat(j, buf0)
    wait(j + 1, buf1, sem1)
    scat(j + 1, buf1)
    plsc.subcore_barrier()

    @pl.when(c == 0)
    def _w0():
        pltpu.sync_copy(acc.at[pl.ds(base, _GRPE)], out0.at[pl.ds(base, _GRPE)])

    @pl.when(c == 1)
    def _w1():
        pltpu.sync_copy(acc.at[pl.ds(base, _GRPE)], out1.at[pl.ds(base, _GRPE)])


_GRPA = ACC_E2N // 16


@functools.partial(
    pl.kernel,
    out_type=(jax.ShapeDtypeStruct((OUT_E2N, D), jnp.float32),
              jax.ShapeDtypeStruct((OUT_E2N, D), jnp.float32)),
    mesh=_mesh(),
    compiler_params=pltpu.CompilerParams(needs_layout_passes=False),
    scratch_types=[
        pltpu.VMEM((NCHUNK, CL), jnp.int32),
        pltpu.VMEM((NCHUNK, CL), jnp.int32),
        pltpu.VMEM((NCHUNK * CL,), jnp.int32),
        pltpu.VMEM((NCHUNK * CL,), jnp.int32),
        pltpu.VMEM((CL,), jnp.int32),
        pltpu.VMEM((CL,), jnp.int32),
        pltpu.VMEM((CL, D), jnp.float32),
        pltpu.VMEM((CL, D), jnp.float32),
        pltpu.VMEM((64, D), jnp.float32),
        pltpu.VMEM_SHARED((ACC_E2N, D), jnp.float32),
        pltpu.SemaphoreType.DMA,
        pltpu.SemaphoreType.DMA,
    ])
def _seg_e2n(table, gidx_h, sidx_h, out0, out1,
             gidx, sidx, gloc, sloc, sstage0, sstage1,
             buf0, buf1, zbuf, acc, sem0, sem1):
    c = lax.axis_index("c")
    s = lax.axis_index("s")
    wid = s * 2 + c
    pltpu.sync_copy(gidx_h.at[wid], gidx)
    pltpu.sync_copy(sidx_h.at[wid], sidx)
    z16 = jnp.zeros((16,), jnp.float32)
    iota16 = lax.iota(jnp.int32, 16)

    def zb(i, _):
        zbuf[i // 8, pl.ds((i % 8) * 16, 16)] = z16
        return 0

    lax.fori_loop(0, 64 * 8, zb, 0)
    base = s * _GRPA

    def run_half(h):
        for zi in range(5):
            pltpu.sync_copy(zbuf, acc.at[pl.ds(base + zi * 64, 64)])
        pltpu.sync_copy(zbuf.at[pl.ds(0, 8)], acc.at[pl.ds(base + 320, 8)])
        plsc.subcore_barrier()

        def pf(i, _):
            gloc[pl.ds(i * 16, 16)] = lax.bitwise_and(iota16 + i * 16, 1023)
            sloc[pl.ds(i * 16, 16)] = NPH + lax.bitwise_and(iota16 + i * 16, 127)
            return 0

        lax.fori_loop(0, NCHUNK * CL // 16, pf, 0)

        def cchunk(j, cnt):
            for g in range(8):
                sv = sidx[j, pl.ds(g * 16, 16)]
                gv = gidx[j, pl.ds(g * 16, 16)]
                t = sv - h * NPH
                m = jnp.logical_and(t >= 0, t < NPH)
                plsc.store_compressed(sloc.at[pl.ds(cnt, 16)], t, mask=m)
                plsc.store_compressed(gloc.at[pl.ds(cnt, 16)], gv, mask=m)
                cnt = cnt + plsc.all_reduce_population_count(m)[0]
            return cnt

        cnt = lax.fori_loop(0, NCHUNK, cchunk, 0)
        nt = jnp.maximum((cnt + CL - 1) // CL, 2)

        def startg(j, buf, sem):
            pltpu.async_copy(table.at[gloc.at[pl.ds(j * CL, CL)]], buf, sem)

        def waitg(j, buf, sem):
            pltpu.make_async_copy(
                table.at[gloc.at[pl.ds(j * CL, CL)]], buf, sem).wait()

        def scats(j, buf, stage):
            for g in range(8):
                stage[pl.ds(g * 16, 16)] = sloc[pl.ds(j * CL + g * 16, 16)]
            pltpu.sync_copy(buf, acc.at[stage], add=True)

        startg(0, buf0, sem0)
        startg(1, buf1, sem1)

        def body(i, _):
            @pl.when(i % 2 == 0)
            def _e():
                waitg(i, buf0, sem0)
                scats(i, buf0, sstage0)

                @pl.when(i + 2 <= nt - 1)
                def _s():
                    startg(i + 2, buf0, sem0)

            @pl.when(i % 2 == 1)
            def _o():
                waitg(i, buf1, sem1)
                scats(i, buf1, sstage1)

                @pl.when(i + 2 <= nt - 1)
                def _s():
                    startg(i + 2, buf1, sem1)

            return 0

        lax.fori_loop(0, nt - 1, body, 0)

        @pl.when((nt - 1) % 2 == 0)
        def _ee():
            waitg(nt - 1, buf0, sem0)
            scats(nt - 1, buf0, sstage0)

        @pl.when((nt - 1) % 2 == 1)
        def _eo():
            waitg(nt - 1, buf1, sem1)
            scats(nt - 1, buf1, sstage1)

        plsc.subcore_barrier()
        ob = h * NPH + base

        @pl.when(c == 0)
        def _w0():
            pltpu.sync_copy(acc.at[pl.ds(base, _GRPA)], out0.at[pl.ds(ob, _GRPA)])

        @pl.when(c == 1)
        def _w1():
            pltpu.sync_copy(acc.at[pl.ds(base, _GRPA)], out1.at[pl.ds(ob, _GRPA)])

        plsc.subcore_barrier()

    run_half(0)
    run_half(1)


_BR = 512


def _colsum_body(a_ref, o_ref):
    o_ref[...] = jnp.sum(a_ref[...], axis=0, keepdims=True)


def _colsum(N):
    return pl.pallas_call(
        _colsum_body,
        grid=(N // 1024,),
        in_specs=[pl.BlockSpec((NW, 1024), lambda i: (0, i))],
        out_specs=pl.BlockSpec((1, 1024), lambda i: (0, i)),
        out_shape=jax.ShapeDtypeStruct((1, N), jnp.float32))


def _mm_body(x_ref, s_ref, w_ref, o_ref):
    o_ref[...] = jnp.dot(x_ref[...] * s_ref[...], w_ref[...],
                         preferred_element_type=jnp.float32)


def _scaled_matmul(N):
    return pl.pallas_call(
        _mm_body,
        grid=(N // _BR,),
        in_specs=[pl.BlockSpec((_BR, D), lambda i: (i, 0)),
                  pl.BlockSpec((_BR, D), lambda i: (i, 0)),
                  pl.BlockSpec((D, D), lambda i: (0, 0))],
        out_specs=pl.BlockSpec((_BR, D), lambda i: (i, 0)),
        out_shape=jax.ShapeDtypeStruct((N, D), jnp.float32))


def _fuse_body(a0_ref, a1_ref, dinv_ref, b_ref, s_ref, w_ref, x_ref, y_ref):
    x = jnp.maximum((a0_ref[...] + a1_ref[...]) * dinv_ref[...] + b_ref[...], 0.0)
    x_ref[...] = x
    y_ref[...] = jnp.dot(x * s_ref[...], w_ref[...],
                         preferred_element_type=jnp.float32)


def _fuse(N):
    return pl.pallas_call(
        _fuse_body,
        grid=(N // _BR,),
        in_specs=[pl.BlockSpec((_BR, D), lambda i: (i, 0)),
                  pl.BlockSpec((_BR, D), lambda i: (i, 0)),
                  pl.BlockSpec((_BR, D), lambda i: (i, 0)),
                  pl.BlockSpec((1, D), lambda i: (0, 0)),
                  pl.BlockSpec((_BR, D), lambda i: (i, 0)),
                  pl.BlockSpec((D, D), lambda i: (0, 0))],
        out_specs=[pl.BlockSpec((_BR, D), lambda i: (i, 0)),
                   pl.BlockSpec((_BR, D), lambda i: (i, 0))],
        out_shape=[jax.ShapeDtypeStruct((N, D), jnp.float32),
                   jax.ShapeDtypeStruct((N, D), jnp.float32)])


def _fin_body(a0_ref, a1_ref, dinv_ref, b_ref, x_ref):
    x_ref[...] = jnp.maximum(
        (a0_ref[...] + a1_ref[...]) * dinv_ref[...] + b_ref[...], 0.0)


def _final(N):
    return pl.pallas_call(
        _fin_body,
        grid=(N // _BR,),
        in_specs=[pl.BlockSpec((_BR, D), lambda i: (i, 0)),
                  pl.BlockSpec((_BR, D), lambda i: (i, 0)),
                  pl.BlockSpec((_BR, D), lambda i: (i, 0)),
                  pl.BlockSpec((1, D), lambda i: (0, 0))],
        out_specs=pl.BlockSpec((_BR, D), lambda i: (i, 0)),
        out_shape=jax.ShapeDtypeStruct((N, D), jnp.float32))


_colsum_n = _colsum(NP)
_colsum_e = _colsum(EP)
_mm_n = _scaled_matmul(NP)
_fuse_e = _fuse(EP)
_fuse_n = _fuse(NP)
_final_n = _final(NP)


def kernel(x_0, W0, W1, b01, b10, rows, cols):
    pad = NW * NCHUNK * CL - NNZ
    pid = jnp.arange(pad, dtype=jnp.int32)
    rows_p = jnp.concatenate([rows, N_NODES + pid % (NP - N_NODES)])
    cols_p = jnp.concatenate([cols, N_EDGES + pid % (EP - N_EDGES)])
    rows_t = rows_p.reshape(NW, NCHUNK, CL)
    cols_t = cols_p.reshape(NW, NCHUNK, CL)
    x0p = jnp.pad(x_0, ((0, NP - N_NODES), (0, 0)))

    cn_p, ce_p = _s1_counts(rows_t, cols_t)
    cn = _colsum_n(cn_p.reshape(NW, NP)).reshape(NP)
    ce = _colsum_e(ce_p.reshape(NW, EP)).reshape(EP)
    d0_p, d1_p = _s2_dsums(rows_t, cols_t, cn, ce)
    d0 = _colsum_n(d0_p.reshape(NW, NP)).reshape(NP)
    d1 = _colsum_e(d1_p.reshape(NW, EP)).reshape(EP)
    nc_b, ec_b, d0i_b, d1i_b = _s3_scales(cn, ce, d0, d1)

    m = _mm_n(x0p, nc_b, W0[0])
    a0, a1 = _seg_n2e(m, rows_t, cols_t)
    _x1, y = _fuse_e(a0, a1, d1i_b, b01[0], ec_b, W1[0])
    n0, n1 = _seg_e2n(y, cols_t, rows_t)
    _x0, m2 = _fuse_n(n0, n1, d0i_b, b10[0], nc_b, W0[1])
    a0, a1 = _seg_n2e(m2, rows_t, cols_t)
    x1f, y2 = _fuse_e(a0, a1, d1i_b, b01[1], ec_b, W1[1])
    n0, n1 = _seg_e2n(y2, cols_t, rows_t)
    x0f = _final_n(n0, n1, d0i_b, b10[1])

    return (x0f[:N_NODES], x1f[:N_EDGES])
